# baseline (speedup 1.0000x reference)
import functools

import jax
import jax.numpy as jnp
from jax import lax
from jax.experimental import pallas as pl
from jax.experimental.pallas import tpu as pltpu
from jax.experimental.pallas import tpu_sc as plsc

_N = 10000
_E = 160000
_DIN = 128
_H0 = 64
_H1 = 64
_FCH = 128

_BLK = 256
_NPAD = 10240
_NB = _NPAD // _BLK
_TW = 32
_PROWS = _NB * _TW * _BLK
_JUNK = 4096
_CH = 128
_EPAD = 163840
_NKEY = 10240
_RW = 16
_REPW = _EPAD // _RW
_KPW = _NKEY // _RW


def _sc_rank(ts_pad, src_p, dst_p):
    mesh = plsc.VectorSubcoreMesh(core_axis_name="c", subcore_axis_name="s",
                                  num_cores=1)
    i32 = jnp.int32

    @functools.partial(
        pl.kernel,
        out_type=[jax.ShapeDtypeStruct((_EPAD,), i32),
                  jax.ShapeDtypeStruct((_NKEY,), i32),
                  jax.ShapeDtypeStruct((_EPAD,), i32)],
        mesh=mesh,
        scratch_types=[
            pltpu.VMEM((_NKEY,), i32),
            pltpu.VMEM((_REPW,), i32),
            pltpu.VMEM((_REPW,), i32),
            pltpu.VMEM((_REPW,), i32),
            pltpu.VMEM((_NKEY,), i32),
            pltpu.VMEM((_REPW,), i32),
            pltpu.VMEM((_RW, _KPW), i32),
            pltpu.VMEM((_KPW,), i32),
            pltpu.VMEM_SHARED((_RW, _NKEY), i32),
            pltpu.SemaphoreType.DMA,
        ],
        compiler_params=pltpu.CompilerParams(use_tc_tiling_on_sc=False,
                                             needs_layout_passes=False),
    )
    def k(ts_hbm, src_hbm, dst_hbm, pos_hbm, deg_hbm, key_hbm,
          tsv, srcv, dstv, kv, hist, posv, colb, acc, shared, sem):
        s = lax.axis_index("s")
        base = s * _REPW
        zeros16 = jnp.zeros((16,), i32)

        def zero_hist(j, c):
            hist[pl.ds(j * 16, 16)] = zeros16
            return c
        lax.fori_loop(0, _NKEY // 16, zero_hist, 0)

        pltpu.sync_copy(ts_hbm, tsv)
        pltpu.sync_copy(src_hbm.at[pl.ds(base, _REPW)], srcv)
        pltpu.sync_copy(dst_hbm.at[pl.ds(base, _REPW)], dstv)

        def p1(i, c):
            sl = pl.ds(i * 16, 16)
            sv = srcv[sl]
            dv = dstv[sl]
            ts_s = plsc.load_gather(tsv, [sv])
            ts_d = plsc.load_gather(tsv, [dv])
            kvec = jnp.where(ts_s <= ts_d, dv, _N)
            kv[sl] = kvec
            cnt, last = plsc.scan_count(kvec)
            plsc.addupdate_scatter(hist, [kvec], cnt, mask=last)
            return c
        lax.fori_loop(0, _REPW // 16, p1, 0)
        pltpu.sync_copy(kv, key_hbm.at[pl.ds(base, _REPW)])

        pltpu.sync_copy(hist, shared.at[s])
        plsc.subcore_barrier()

        col0 = s * _KPW
        pltpu.sync_copy(shared.at[:, pl.ds(col0, _KPW)], colb)
        for j in range(_KPW // 16):
            acc[pl.ds(j * 16, 16)] = zeros16

        def p2(w, c):
            for j in range(_KPW // 16):
                b = acc[pl.ds(j * 16, 16)]
                r = colb[w, pl.ds(j * 16, 16)]
                colb[w, pl.ds(j * 16, 16)] = b
                acc[pl.ds(j * 16, 16)] = b + r
            return c
        lax.fori_loop(0, _RW, p2, 0)
        pltpu.sync_copy(colb, shared.at[:, pl.ds(col0, _KPW)])
        pltpu.sync_copy(acc, deg_hbm.at[pl.ds(col0, _KPW)])
        plsc.subcore_barrier()

        pltpu.sync_copy(shared.at[s], hist)

        def p3(i, c):
            kvec = kv[pl.ds(i * 16, 16)]
            cnt, last = plsc.scan_count(kvec)
            b = plsc.load_gather(hist, [kvec])
            posv[pl.ds(i * 16, 16)] = b + cnt - 1
            plsc.addupdate_scatter(hist, [kvec], cnt, mask=last)
            return c
        lax.fori_loop(0, _REPW // 16, p3, 0)
        pltpu.sync_copy(posv, pos_hbm.at[pl.ds(base, _REPW)])

    return k(ts_pad, src_p, dst_p)


def _sc_build_panel(table, src_idx, tgt_idx, d):
    mesh = plsc.VectorSubcoreMesh(core_axis_name="c", subcore_axis_name="s")
    nc, ns = mesh.num_cores, mesh.num_subcores
    nw = nc * ns
    epw = _EPAD // nw
    nchunks = epw // _CH
    nbuf = 4
    nq = nchunks // nbuf

    @functools.partial(
        pl.kernel,
        out_type=jax.ShapeDtypeStruct((_PROWS + _JUNK, d), jnp.float32),
        mesh=mesh,
        scratch_types=[
            pltpu.VMEM((nchunks, _CH), jnp.int32),
            pltpu.VMEM((nchunks, _CH), jnp.int32),
        ] + [pltpu.VMEM((_CH, d), jnp.float32) for _ in range(nbuf)]
          + [pltpu.SemaphoreType.DMA for _ in range(2 * nbuf)],
        compiler_params=pltpu.CompilerParams(use_tc_tiling_on_sc=False),
    )
    def k(table_hbm, src_hbm, tgt_hbm, out_hbm, srcv, tgtv, *bufsem):
        rows = bufsem[:nbuf]
        semg = bufsem[nbuf:2 * nbuf]
        sems = bufsem[2 * nbuf:]
        wid = lax.axis_index("s") * nc + lax.axis_index("c")
        pltpu.sync_copy(src_hbm.at[wid], srcv)
        pltpu.sync_copy(tgt_hbm.at[wid], tgtv)

        for b in range(nbuf):
            pltpu.async_copy(table_hbm.at[srcv.at[b]], rows[b], semg[b])

        def quad(j, carry):
            i0 = j * nbuf
            for b in range(nbuf):
                pltpu.make_async_copy(
                    table_hbm.at[srcv.at[i0 + b]], rows[b], semg[b]).wait()
                pltpu.async_copy(rows[b], out_hbm.at[tgtv.at[i0 + b]],
                                 sems[b])
            for b in range(nbuf):
                pltpu.make_async_copy(
                    rows[b], out_hbm.at[tgtv.at[i0 + b]], sems[b]).wait()

                @pl.when(j < nq - 1)
                def _():
                    pltpu.async_copy(table_hbm.at[srcv.at[i0 + b + nbuf]],
                                     rows[b], semg[b])
            return carry

        lax.fori_loop(0, nq, quad, 0)

    return k(table, src_idx.reshape(nw, nchunks, _CH),
             tgt_idx.reshape(nw, nchunks, _CH))


def _tc_conv_window(panel, degcol, h, c, scal, wih, whh, brow, xbrow,
                    d, hdim):
    g4 = 4 * hdim

    def body(scal_ref, panel_ref, deg_ref, hin_ref, cin_ref, wih_ref,
             whh_ref, b_ref, xb_ref, hout_ref, cout_ref, gx_ref):
        twc = scal_ref[0]
        wbase = scal_ref[1]
        degb = deg_ref[...]
        wihv = wih_ref[...]
        whhv = whh_ref[...]
        bias = b_ref[...]

        pad8 = jnp.dot(jnp.full((8, d), -999.0, jnp.float32), wihv,
                       preferred_element_type=jnp.float32)
        padg = pad8[0:1, :]
        xbw8 = jnp.dot(jnp.broadcast_to(xb_ref[...], (8, d)), wihv,
                       preferred_element_type=jnp.float32)
        xbw = xbw8[0:1, :]

        gx_ref[...] = jnp.dot(panel_ref[...], wihv,
                              preferred_element_type=jnp.float32)

        def step(tt, hc):
            hcur, ccur = hc
            gxs = gx_ref[pl.ds(tt * _BLK, _BLK), :]
            valid = (wbase + tt) < degb
            gates = (jnp.where(valid, gxs + xbw, padg)
                     + jnp.dot(hcur, whhv, preferred_element_type=jnp.float32)
                     + bias)
            i_ = jax.nn.sigmoid(gates[:, 0 * hdim:1 * hdim])
            f_ = jax.nn.sigmoid(gates[:, 1 * hdim:2 * hdim])
            g_ = jnp.tanh(gates[:, 2 * hdim:3 * hdim])
            o_ = jax.nn.sigmoid(gates[:, 3 * hdim:4 * hdim])
            cn = f_ * ccur + i_ * g_
            hn = o_ * jnp.tanh(cn)
            return (hn, cn)

        hfin, cfin = lax.fori_loop(0, twc, step, (hin_ref[...], cin_ref[...]))
        hout_ref[...] = hfin
        cout_ref[...] = cfin

    return pl.pallas_call(
        body,
        grid=(_NB,),
        in_specs=[
            pl.BlockSpec(memory_space=pltpu.SMEM),
            pl.BlockSpec((_TW * _BLK, d), lambda b: (b, 0)),
            pl.BlockSpec((_BLK, 1), lambda b: (b, 0)),
            pl.BlockSpec((_BLK, hdim), lambda b: (b, 0)),
            pl.BlockSpec((_BLK, hdim), lambda b: (b, 0)),
            pl.BlockSpec((d, g4), lambda b: (0, 0)),
            pl.BlockSpec((hdim, g4), lambda b: (0, 0)),
            pl.BlockSpec((1, g4), lambda b: (0, 0)),
            pl.BlockSpec((1, d), lambda b: (0, 0)),
        ],
        out_specs=[
            pl.BlockSpec((_BLK, hdim), lambda b: (b, 0)),
            pl.BlockSpec((_BLK, hdim), lambda b: (b, 0)),
        ],
        out_shape=[jax.ShapeDtypeStruct((_NPAD, hdim), jnp.float32)] * 2,
        scratch_shapes=[pltpu.VMEM((_TW * _BLK, g4), jnp.float32)],
    )(scal, panel, degcol, h, c, wih, whh, brow, xbrow)


def _tc_fc(h1, cbrow, w1, b1row, w2, b2row):
    def body(h_ref, cb_ref, w1_ref, b1_ref, w2_ref, b2_ref, o_ref):
        hb = h_ref[...] + cb_ref[...]
        a = jnp.dot(hb, w1_ref[...], preferred_element_type=jnp.float32) + b1_ref[...]
        a = jnp.maximum(a, 0.0)
        o_ref[...] = (jnp.dot(a, w2_ref[...], preferred_element_type=jnp.float32)
                      + b2_ref[...])

    return pl.pallas_call(
        body,
        grid=(_NB,),
        in_specs=[
            pl.BlockSpec((_BLK, _H1), lambda b: (b, 0)),
            pl.BlockSpec((1, _H1), lambda b: (0, 0)),
            pl.BlockSpec((_H1, _FCH), lambda b: (0, 0)),
            pl.BlockSpec((1, _FCH), lambda b: (0, 0)),
            pl.BlockSpec((_FCH, 128), lambda b: (0, 0)),
            pl.BlockSpec((1, 128), lambda b: (0, 0)),
        ],
        out_specs=pl.BlockSpec((_BLK, 128), lambda b: (b, 0)),
        out_shape=jax.ShapeDtypeStruct((_NPAD, 128), jnp.float32),
    )(h1, cbrow, w1, b1row, w2, b2row)


def kernel(x, node_ts, edge_index, conv0_W_ih, conv0_W_hh, conv0_b_ih,
           conv0_b_hh, conv0_bias, conv1_W_ih, conv1_W_hh, conv1_b_ih,
           conv1_b_hh, conv1_bias, fc1_W, fc1_b, fc2_W, fc2_b):
    f32 = jnp.float32
    src, dst = edge_index[0], edge_index[1]

    pad_e = _EPAD - _E
    src_p = jnp.concatenate([src.astype(jnp.int32),
                             jnp.zeros((pad_e,), jnp.int32)])
    dst_p = jnp.concatenate([dst.astype(jnp.int32),
                             jnp.full((pad_e,), _NKEY - 1, jnp.int32)])
    ts_pad = jnp.concatenate([node_ts.astype(jnp.int32),
                              jnp.zeros((_NKEY - _N,), jnp.int32)])
    pos_p, deg_full, key_p = _sc_rank(ts_pad, src_p, dst_p)
    deg = deg_full[:_N]
    max_deg = jnp.maximum(jnp.max(deg), 1)
    blocki = key_p // _BLK
    dloc = key_p % _BLK
    junk_tgt = _PROWS + (jnp.arange(_EPAD, dtype=jnp.int32) % _JUNK)

    degcol = jnp.concatenate(
        [deg, jnp.zeros((_NPAD - _N,), jnp.int32)]).reshape(_NPAD, 1)

    def run_conv(table, d, hdim, wih, whh, brow, xbrow):
        h = jnp.zeros((_NPAD, hdim), f32)
        c = jnp.zeros((_NPAD, hdim), f32)

        def cond(st):
            w, _, _ = st
            return w * _TW < max_deg

        def body(st):
            w, h, c = st
            wbase = w * _TW
            in_win = (key_p < _N) & (pos_p >= wbase) & (pos_p < wbase + _TW)
            tgt = jnp.where(
                in_win,
                blocki * (_TW * _BLK) + (pos_p - wbase) * _BLK + dloc,
                junk_tgt)
            panel = _sc_build_panel(table, src_p, tgt, d)
            twc = jnp.minimum(max_deg - wbase, _TW)
            scal = jnp.stack([twc, wbase]).astype(jnp.int32)
            h, c = _tc_conv_window(panel, degcol, h, c, scal, wih, whh,
                                   brow, xbrow, d, hdim)
            return (w + 1, h, c)

        _, h, _ = lax.while_loop(cond, body, (jnp.int32(0), h, c))
        return h

    w0ih = conv0_W_ih.T
    w0hh = conv0_W_hh.T
    b0 = (conv0_b_ih + conv0_b_hh).reshape(1, 4 * _H0)
    xb0 = jnp.zeros((1, _DIN), f32)
    h0 = run_conv(x, _DIN, _H0, w0ih, w0hh, b0, xb0)

    w1ih = conv1_W_ih.T
    w1hh = conv1_W_hh.T
    b1 = (conv1_b_ih + conv1_b_hh).reshape(1, 4 * _H1)
    xb1 = conv0_bias.reshape(1, _H0)
    h1 = run_conv(h0, _H0, _H1, w1ih, w1hh, b1, xb1)

    w2pad = jnp.zeros((_FCH, 128), f32).at[:, :2].set(fc2_W.T)
    b2row = jnp.zeros((1, 128), f32).at[:, :2].set(fc2_b.reshape(1, 2))
    out = _tc_fc(h1, conv1_bias.reshape(1, _H1), fc1_W.T,
                 fc1_b.reshape(1, _FCH), w2pad, b2row)
    return out[:_N, :2]


# speedup vs baseline: 1.1819x; 1.1819x over previous
import functools

import jax
import jax.numpy as jnp
from jax import lax
from jax.experimental import pallas as pl
from jax.experimental.pallas import tpu as pltpu
from jax.experimental.pallas import tpu_sc as plsc

_N = 10000
_E = 160000
_DIN = 128
_H0 = 64
_H1 = 64
_FCH = 128

_BLK = 512
_NPAD = 10240
_NB = _NPAD // _BLK
_TW = 32
_PROWS = _NB * _TW * _BLK
_JUNK = 4096
_CH = 128
_EPAD = 163840
_NKEY = 10240
_RW = 16
_REPW = _EPAD // _RW
_KPW = _NKEY // _RW


def _sc_rank(ts_pad, src_p, dst_p):
    mesh = plsc.VectorSubcoreMesh(core_axis_name="c", subcore_axis_name="s",
                                  num_cores=1)
    i32 = jnp.int32

    @functools.partial(
        pl.kernel,
        out_type=[jax.ShapeDtypeStruct((_EPAD,), i32),
                  jax.ShapeDtypeStruct((_NKEY,), i32),
                  jax.ShapeDtypeStruct((_EPAD,), i32)],
        mesh=mesh,
        scratch_types=[
            pltpu.VMEM((_NKEY,), i32),
            pltpu.VMEM((_REPW,), i32),
            pltpu.VMEM((_REPW,), i32),
            pltpu.VMEM((_REPW,), i32),
            pltpu.VMEM((_NKEY,), i32),
            pltpu.VMEM((_REPW,), i32),
            pltpu.VMEM((_RW, _KPW), i32),
            pltpu.VMEM((_KPW,), i32),
            pltpu.VMEM_SHARED((_RW, _NKEY), i32),
            pltpu.SemaphoreType.DMA,
        ],
        compiler_params=pltpu.CompilerParams(use_tc_tiling_on_sc=False,
                                             needs_layout_passes=False),
    )
    def k(ts_hbm, src_hbm, dst_hbm, pos_hbm, deg_hbm, key_hbm,
          tsv, srcv, dstv, kv, hist, posv, colb, acc, shared, sem):
        s = lax.axis_index("s")
        base = s * _REPW
        zeros16 = jnp.zeros((16,), i32)

        def zero_hist(j, c):
            hist[pl.ds(j * 16, 16)] = zeros16
            return c
        lax.fori_loop(0, _NKEY // 16, zero_hist, 0)

        pltpu.sync_copy(ts_hbm, tsv)
        pltpu.sync_copy(src_hbm.at[pl.ds(base, _REPW)], srcv)
        pltpu.sync_copy(dst_hbm.at[pl.ds(base, _REPW)], dstv)

        def p1(i, c):
            sl = pl.ds(i * 16, 16)
            sv = srcv[sl]
            dv = dstv[sl]
            ts_s = plsc.load_gather(tsv, [sv])
            ts_d = plsc.load_gather(tsv, [dv])
            kvec = jnp.where(ts_s <= ts_d, dv, _N)
            kv[sl] = kvec
            cnt, last = plsc.scan_count(kvec)
            plsc.addupdate_scatter(hist, [kvec], cnt, mask=last)
            return c
        lax.fori_loop(0, _REPW // 16, p1, 0)
        pltpu.sync_copy(kv, key_hbm.at[pl.ds(base, _REPW)])

        pltpu.sync_copy(hist, shared.at[s])
        plsc.subcore_barrier()

        col0 = s * _KPW
        pltpu.sync_copy(shared.at[:, pl.ds(col0, _KPW)], colb)
        for j in range(_KPW // 16):
            acc[pl.ds(j * 16, 16)] = zeros16

        def p2(w, c):
            for j in range(_KPW // 16):
                b = acc[pl.ds(j * 16, 16)]
                r = colb[w, pl.ds(j * 16, 16)]
                colb[w, pl.ds(j * 16, 16)] = b
                acc[pl.ds(j * 16, 16)] = b + r
            return c
        lax.fori_loop(0, _RW, p2, 0)
        pltpu.sync_copy(colb, shared.at[:, pl.ds(col0, _KPW)])
        pltpu.sync_copy(acc, deg_hbm.at[pl.ds(col0, _KPW)])
        plsc.subcore_barrier()

        pltpu.sync_copy(shared.at[s], hist)

        def p3(i, c):
            kvec = kv[pl.ds(i * 16, 16)]
            cnt, last = plsc.scan_count(kvec)
            b = plsc.load_gather(hist, [kvec])
            posv[pl.ds(i * 16, 16)] = b + cnt - 1
            plsc.addupdate_scatter(hist, [kvec], cnt, mask=last)
            return c
        lax.fori_loop(0, _REPW // 16, p3, 0)
        pltpu.sync_copy(posv, pos_hbm.at[pl.ds(base, _REPW)])

    return k(ts_pad, src_p, dst_p)


def _sc_build_panel(table, src_idx, tgt_idx, d):
    mesh = plsc.VectorSubcoreMesh(core_axis_name="c", subcore_axis_name="s")
    nc, ns = mesh.num_cores, mesh.num_subcores
    nw = nc * ns
    epw = _EPAD // nw
    nchunks = epw // _CH
    nbuf = 4
    nq = nchunks // nbuf

    @functools.partial(
        pl.kernel,
        out_type=jax.ShapeDtypeStruct((_PROWS + _JUNK, d), jnp.float32),
        mesh=mesh,
        scratch_types=[
            pltpu.VMEM((nchunks, _CH), jnp.int32),
            pltpu.VMEM((nchunks, _CH), jnp.int32),
        ] + [pltpu.VMEM((_CH, d), jnp.float32) for _ in range(nbuf)]
          + [pltpu.SemaphoreType.DMA for _ in range(2 * nbuf)],
        compiler_params=pltpu.CompilerParams(use_tc_tiling_on_sc=False),
    )
    def k(table_hbm, src_hbm, tgt_hbm, out_hbm, srcv, tgtv, *bufsem):
        rows = bufsem[:nbuf]
        semg = bufsem[nbuf:2 * nbuf]
        sems = bufsem[2 * nbuf:]
        wid = lax.axis_index("s") * nc + lax.axis_index("c")
        pltpu.sync_copy(src_hbm.at[wid], srcv)
        pltpu.sync_copy(tgt_hbm.at[wid], tgtv)

        for b in range(nbuf):
            pltpu.async_copy(table_hbm.at[srcv.at[b]], rows[b], semg[b])

        def quad(j, carry):
            i0 = j * nbuf
            for b in range(nbuf):
                pltpu.make_async_copy(
                    table_hbm.at[srcv.at[i0 + b]], rows[b], semg[b]).wait()
                pltpu.async_copy(rows[b], out_hbm.at[tgtv.at[i0 + b]],
                                 sems[b])
            for b in range(nbuf):
                pltpu.make_async_copy(
                    rows[b], out_hbm.at[tgtv.at[i0 + b]], sems[b]).wait()

                @pl.when(j < nq - 1)
                def _():
                    pltpu.async_copy(table_hbm.at[srcv.at[i0 + b + nbuf]],
                                     rows[b], semg[b])
            return carry

        lax.fori_loop(0, nq, quad, 0)

    return k(table, src_idx.reshape(nw, nchunks, _CH),
             tgt_idx.reshape(nw, nchunks, _CH))


def _tc_conv_window(panel, degcol, h, c, scal, wih, whh, brow, xbrow,
                    d, hdim):
    g4 = 4 * hdim

    def body(scal_ref, panel_ref, deg_ref, hin_ref, cin_ref, wih_ref,
             whh_ref, b_ref, xb_ref, hout_ref, cout_ref, gx_ref):
        twc = scal_ref[0]
        wbase = scal_ref[1]
        degb = deg_ref[...]
        wihv = wih_ref[...]
        whhv = whh_ref[...]
        bias = b_ref[...]

        pad8 = jnp.dot(jnp.full((8, d), -999.0, jnp.float32), wihv,
                       preferred_element_type=jnp.float32)
        padg = pad8[0:1, :]
        xbw8 = jnp.dot(jnp.broadcast_to(xb_ref[...], (8, d)), wihv,
                       preferred_element_type=jnp.float32)
        xbw = xbw8[0:1, :]

        gx_ref[...] = jnp.dot(panel_ref[...], wihv,
                              preferred_element_type=jnp.float32)

        def step(tt, hc):
            hcur, ccur = hc
            gxs = gx_ref[pl.ds(tt * _BLK, _BLK), :]
            valid = (wbase + tt) < degb
            gates = (jnp.where(valid, gxs + xbw, padg)
                     + jnp.dot(hcur, whhv, preferred_element_type=jnp.float32)
                     + bias)
            i_ = jax.nn.sigmoid(gates[:, 0 * hdim:1 * hdim])
            f_ = jax.nn.sigmoid(gates[:, 1 * hdim:2 * hdim])
            g_ = jnp.tanh(gates[:, 2 * hdim:3 * hdim])
            o_ = jax.nn.sigmoid(gates[:, 3 * hdim:4 * hdim])
            cn = f_ * ccur + i_ * g_
            hn = o_ * jnp.tanh(cn)
            return (hn, cn)

        hfin, cfin = lax.fori_loop(0, twc, step, (hin_ref[...], cin_ref[...]))
        hout_ref[...] = hfin
        cout_ref[...] = cfin

    return pl.pallas_call(
        body,
        grid=(_NB,),
        in_specs=[
            pl.BlockSpec(memory_space=pltpu.SMEM),
            pl.BlockSpec((_TW * _BLK, d), lambda b: (b, 0)),
            pl.BlockSpec((_BLK, 1), lambda b: (b, 0)),
            pl.BlockSpec((_BLK, hdim), lambda b: (b, 0)),
            pl.BlockSpec((_BLK, hdim), lambda b: (b, 0)),
            pl.BlockSpec((d, g4), lambda b: (0, 0)),
            pl.BlockSpec((hdim, g4), lambda b: (0, 0)),
            pl.BlockSpec((1, g4), lambda b: (0, 0)),
            pl.BlockSpec((1, d), lambda b: (0, 0)),
        ],
        out_specs=[
            pl.BlockSpec((_BLK, hdim), lambda b: (b, 0)),
            pl.BlockSpec((_BLK, hdim), lambda b: (b, 0)),
        ],
        out_shape=[jax.ShapeDtypeStruct((_NPAD, hdim), jnp.float32)] * 2,
        scratch_shapes=[pltpu.VMEM((_TW * _BLK, g4), jnp.float32)],
    )(scal, panel, degcol, h, c, wih, whh, brow, xbrow)


def _tc_fc(h1, cbrow, w1, b1row, w2, b2row):
    def body(h_ref, cb_ref, w1_ref, b1_ref, w2_ref, b2_ref, o_ref):
        hb = h_ref[...] + cb_ref[...]
        a = jnp.dot(hb, w1_ref[...], preferred_element_type=jnp.float32) + b1_ref[...]
        a = jnp.maximum(a, 0.0)
        o_ref[...] = (jnp.dot(a, w2_ref[...], preferred_element_type=jnp.float32)
                      + b2_ref[...])

    return pl.pallas_call(
        body,
        grid=(_NB,),
        in_specs=[
            pl.BlockSpec((_BLK, _H1), lambda b: (b, 0)),
            pl.BlockSpec((1, _H1), lambda b: (0, 0)),
            pl.BlockSpec((_H1, _FCH), lambda b: (0, 0)),
            pl.BlockSpec((1, _FCH), lambda b: (0, 0)),
            pl.BlockSpec((_FCH, 128), lambda b: (0, 0)),
            pl.BlockSpec((1, 128), lambda b: (0, 0)),
        ],
        out_specs=pl.BlockSpec((_BLK, 128), lambda b: (b, 0)),
        out_shape=jax.ShapeDtypeStruct((_NPAD, 128), jnp.float32),
    )(h1, cbrow, w1, b1row, w2, b2row)


def kernel(x, node_ts, edge_index, conv0_W_ih, conv0_W_hh, conv0_b_ih,
           conv0_b_hh, conv0_bias, conv1_W_ih, conv1_W_hh, conv1_b_ih,
           conv1_b_hh, conv1_bias, fc1_W, fc1_b, fc2_W, fc2_b):
    f32 = jnp.float32
    src, dst = edge_index[0], edge_index[1]

    pad_e = _EPAD - _E
    src_p = jnp.concatenate([src.astype(jnp.int32),
                             jnp.zeros((pad_e,), jnp.int32)])
    dst_p = jnp.concatenate([dst.astype(jnp.int32),
                             jnp.full((pad_e,), _NKEY - 1, jnp.int32)])
    ts_pad = jnp.concatenate([node_ts.astype(jnp.int32),
                              jnp.zeros((_NKEY - _N,), jnp.int32)])
    pos_p, deg_full, key_p = _sc_rank(ts_pad, src_p, dst_p)
    deg = deg_full[:_N]
    max_deg = jnp.maximum(jnp.max(deg), 1)
    blocki = key_p // _BLK
    dloc = key_p % _BLK
    junk_tgt = _PROWS + (jnp.arange(_EPAD, dtype=jnp.int32) % _JUNK)

    degcol = jnp.concatenate(
        [deg, jnp.zeros((_NPAD - _N,), jnp.int32)]).reshape(_NPAD, 1)

    def run_conv(table, d, hdim, wih, whh, brow, xbrow):
        h = jnp.zeros((_NPAD, hdim), f32)
        c = jnp.zeros((_NPAD, hdim), f32)

        def cond(st):
            w, _, _ = st
            return w * _TW < max_deg

        def body(st):
            w, h, c = st
            wbase = w * _TW
            in_win = (key_p < _N) & (pos_p >= wbase) & (pos_p < wbase + _TW)
            tgt = jnp.where(
                in_win,
                blocki * (_TW * _BLK) + (pos_p - wbase) * _BLK + dloc,
                junk_tgt)
            panel = _sc_build_panel(table, src_p, tgt, d)
            twc = jnp.minimum(max_deg - wbase, _TW)
            scal = jnp.stack([twc, wbase]).astype(jnp.int32)
            h, c = _tc_conv_window(panel, degcol, h, c, scal, wih, whh,
                                   brow, xbrow, d, hdim)
            return (w + 1, h, c)

        _, h, _ = lax.while_loop(cond, body, (jnp.int32(0), h, c))
        return h

    w0ih = conv0_W_ih.T
    w0hh = conv0_W_hh.T
    b0 = (conv0_b_ih + conv0_b_hh).reshape(1, 4 * _H0)
    xb0 = jnp.zeros((1, _DIN), f32)
    h0 = run_conv(x, _DIN, _H0, w0ih, w0hh, b0, xb0)

    w1ih = conv1_W_ih.T
    w1hh = conv1_W_hh.T
    b1 = (conv1_b_ih + conv1_b_hh).reshape(1, 4 * _H1)
    xb1 = conv0_bias.reshape(1, _H0)
    h1 = run_conv(h0, _H0, _H1, w1ih, w1hh, b1, xb1)

    w2pad = jnp.zeros((_FCH, 128), f32).at[:, :2].set(fc2_W.T)
    b2row = jnp.zeros((1, 128), f32).at[:, :2].set(fc2_b.reshape(1, 2))
    out = _tc_fc(h1, conv1_bias.reshape(1, _H1), fc1_W.T,
                 fc1_b.reshape(1, _FCH), w2pad, b2row)
    return out[:_N, :2]


# speedup vs baseline: 1.2058x; 1.0203x over previous
import functools

import jax
import jax.numpy as jnp
from jax import lax
from jax.experimental import pallas as pl
from jax.experimental.pallas import tpu as pltpu
from jax.experimental.pallas import tpu_sc as plsc

_N = 10000
_E = 160000
_DIN = 128
_H0 = 64
_H1 = 64
_FCH = 128

_BLK = 640
_NPAD = 10240
_NB = _NPAD // _BLK
_TW = 32
_PROWS = _NB * _TW * _BLK
_JUNK = 4096
_CH = 128
_EPAD = 163840
_NKEY = 10240
_RW = 16
_REPW = _EPAD // _RW
_KPW = _NKEY // _RW


def _sc_rank(ts_pad, src_p, dst_p):
    mesh = plsc.VectorSubcoreMesh(core_axis_name="c", subcore_axis_name="s",
                                  num_cores=1)
    i32 = jnp.int32

    @functools.partial(
        pl.kernel,
        out_type=[jax.ShapeDtypeStruct((_EPAD,), i32),
                  jax.ShapeDtypeStruct((_NKEY,), i32),
                  jax.ShapeDtypeStruct((_EPAD,), i32)],
        mesh=mesh,
        scratch_types=[
            pltpu.VMEM((_NKEY,), i32),
            pltpu.VMEM((_REPW,), i32),
            pltpu.VMEM((_REPW,), i32),
            pltpu.VMEM((_REPW,), i32),
            pltpu.VMEM((_NKEY,), i32),
            pltpu.VMEM((_REPW,), i32),
            pltpu.VMEM((_RW, _KPW), i32),
            pltpu.VMEM((_KPW,), i32),
            pltpu.VMEM_SHARED((_RW, _NKEY), i32),
            pltpu.SemaphoreType.DMA,
        ],
        compiler_params=pltpu.CompilerParams(use_tc_tiling_on_sc=False,
                                             needs_layout_passes=False),
    )
    def k(ts_hbm, src_hbm, dst_hbm, pos_hbm, deg_hbm, key_hbm,
          tsv, srcv, dstv, kv, hist, posv, colb, acc, shared, sem):
        s = lax.axis_index("s")
        base = s * _REPW
        zeros16 = jnp.zeros((16,), i32)

        def zero_hist(j, c):
            hist[pl.ds(j * 16, 16)] = zeros16
            return c
        lax.fori_loop(0, _NKEY // 16, zero_hist, 0)

        pltpu.sync_copy(ts_hbm, tsv)
        pltpu.sync_copy(src_hbm.at[pl.ds(base, _REPW)], srcv)
        pltpu.sync_copy(dst_hbm.at[pl.ds(base, _REPW)], dstv)

        def p1(i, c):
            sl = pl.ds(i * 16, 16)
            sv = srcv[sl]
            dv = dstv[sl]
            ts_s = plsc.load_gather(tsv, [sv])
            ts_d = plsc.load_gather(tsv, [dv])
            kvec = jnp.where(ts_s <= ts_d, dv, _N)
            kv[sl] = kvec
            cnt, last = plsc.scan_count(kvec)
            plsc.addupdate_scatter(hist, [kvec], cnt, mask=last)
            return c
        lax.fori_loop(0, _REPW // 16, p1, 0)
        pltpu.sync_copy(kv, key_hbm.at[pl.ds(base, _REPW)])

        pltpu.sync_copy(hist, shared.at[s])
        plsc.subcore_barrier()

        col0 = s * _KPW
        pltpu.sync_copy(shared.at[:, pl.ds(col0, _KPW)], colb)
        for j in range(_KPW // 16):
            acc[pl.ds(j * 16, 16)] = zeros16

        def p2(w, c):
            for j in range(_KPW // 16):
                b = acc[pl.ds(j * 16, 16)]
                r = colb[w, pl.ds(j * 16, 16)]
                colb[w, pl.ds(j * 16, 16)] = b
                acc[pl.ds(j * 16, 16)] = b + r
            return c
        lax.fori_loop(0, _RW, p2, 0)
        pltpu.sync_copy(colb, shared.at[:, pl.ds(col0, _KPW)])
        pltpu.sync_copy(acc, deg_hbm.at[pl.ds(col0, _KPW)])
        plsc.subcore_barrier()

        pltpu.sync_copy(shared.at[s], hist)

        def p3(i, c):
            kvec = kv[pl.ds(i * 16, 16)]
            cnt, last = plsc.scan_count(kvec)
            b = plsc.load_gather(hist, [kvec])
            posv[pl.ds(i * 16, 16)] = b + cnt - 1
            plsc.addupdate_scatter(hist, [kvec], cnt, mask=last)
            return c
        lax.fori_loop(0, _REPW // 16, p3, 0)
        pltpu.sync_copy(posv, pos_hbm.at[pl.ds(base, _REPW)])

    return k(ts_pad, src_p, dst_p)


def _sc_build_panel(table, src_idx, tgt_idx, d):
    mesh = plsc.VectorSubcoreMesh(core_axis_name="c", subcore_axis_name="s")
    nc, ns = mesh.num_cores, mesh.num_subcores
    nw = nc * ns
    epw = _EPAD // nw
    nchunks = epw // _CH
    nbuf = 4
    nq = nchunks // nbuf

    @functools.partial(
        pl.kernel,
        out_type=jax.ShapeDtypeStruct((_PROWS + _JUNK, d), jnp.float32),
        mesh=mesh,
        scratch_types=[
            pltpu.VMEM((nchunks, _CH), jnp.int32),
            pltpu.VMEM((nchunks, _CH), jnp.int32),
        ] + [pltpu.VMEM((_CH, d), jnp.float32) for _ in range(nbuf)]
          + [pltpu.SemaphoreType.DMA for _ in range(2 * nbuf)],
        compiler_params=pltpu.CompilerParams(use_tc_tiling_on_sc=False),
    )
    def k(table_hbm, src_hbm, tgt_hbm, out_hbm, srcv, tgtv, *bufsem):
        rows = bufsem[:nbuf]
        semg = bufsem[nbuf:2 * nbuf]
        sems = bufsem[2 * nbuf:]
        wid = lax.axis_index("s") * nc + lax.axis_index("c")
        pltpu.sync_copy(src_hbm.at[wid], srcv)
        pltpu.sync_copy(tgt_hbm.at[wid], tgtv)

        for b in range(nbuf):
            pltpu.async_copy(table_hbm.at[srcv.at[b]], rows[b], semg[b])

        def quad(j, carry):
            i0 = j * nbuf
            for b in range(nbuf):
                pltpu.make_async_copy(
                    table_hbm.at[srcv.at[i0 + b]], rows[b], semg[b]).wait()
                pltpu.async_copy(rows[b], out_hbm.at[tgtv.at[i0 + b]],
                                 sems[b])
            for b in range(nbuf):
                pltpu.make_async_copy(
                    rows[b], out_hbm.at[tgtv.at[i0 + b]], sems[b]).wait()

                @pl.when(j < nq - 1)
                def _():
                    pltpu.async_copy(table_hbm.at[srcv.at[i0 + b + nbuf]],
                                     rows[b], semg[b])
            return carry

        lax.fori_loop(0, nq, quad, 0)

    return k(table, src_idx.reshape(nw, nchunks, _CH),
             tgt_idx.reshape(nw, nchunks, _CH))


def _tc_conv_window(panel, degcol, h, c, scal, wih, whh, brow, xbrow,
                    d, hdim):
    g4 = 4 * hdim

    def body(scal_ref, panel_ref, deg_ref, hin_ref, cin_ref, wih_ref,
             whh_ref, b_ref, xb_ref, hout_ref, cout_ref, gx_ref):
        twc = scal_ref[0]
        wbase = scal_ref[1]
        degb = deg_ref[...]
        wihv = wih_ref[...]
        whhv = whh_ref[...]
        bias = b_ref[...]

        pad8 = jnp.dot(jnp.full((8, d), -999.0, jnp.float32), wihv,
                       preferred_element_type=jnp.float32)
        padg = pad8[0:1, :]
        xbw8 = jnp.dot(jnp.broadcast_to(xb_ref[...], (8, d)), wihv,
                       preferred_element_type=jnp.float32)
        xbw = xbw8[0:1, :]

        gx_ref[...] = jnp.dot(panel_ref[...], wihv,
                              preferred_element_type=jnp.float32)

        def step(tt, hc):
            hcur, ccur = hc
            gxs = gx_ref[pl.ds(tt * _BLK, _BLK), :]
            valid = (wbase + tt) < degb
            gates = (jnp.where(valid, gxs + xbw, padg)
                     + jnp.dot(hcur, whhv, preferred_element_type=jnp.float32)
                     + bias)
            i_ = jax.nn.sigmoid(gates[:, 0 * hdim:1 * hdim])
            f_ = jax.nn.sigmoid(gates[:, 1 * hdim:2 * hdim])
            g_ = jnp.tanh(gates[:, 2 * hdim:3 * hdim])
            o_ = jax.nn.sigmoid(gates[:, 3 * hdim:4 * hdim])
            cn = f_ * ccur + i_ * g_
            hn = o_ * jnp.tanh(cn)
            return (hn, cn)

        hfin, cfin = lax.fori_loop(0, twc, step, (hin_ref[...], cin_ref[...]))
        hout_ref[...] = hfin
        cout_ref[...] = cfin

    return pl.pallas_call(
        body,
        grid=(_NB,),
        in_specs=[
            pl.BlockSpec(memory_space=pltpu.SMEM),
            pl.BlockSpec((_TW * _BLK, d), lambda b: (b, 0)),
            pl.BlockSpec((_BLK, 1), lambda b: (b, 0)),
            pl.BlockSpec((_BLK, hdim), lambda b: (b, 0)),
            pl.BlockSpec((_BLK, hdim), lambda b: (b, 0)),
            pl.BlockSpec((d, g4), lambda b: (0, 0)),
            pl.BlockSpec((hdim, g4), lambda b: (0, 0)),
            pl.BlockSpec((1, g4), lambda b: (0, 0)),
            pl.BlockSpec((1, d), lambda b: (0, 0)),
        ],
        out_specs=[
            pl.BlockSpec((_BLK, hdim), lambda b: (b, 0)),
            pl.BlockSpec((_BLK, hdim), lambda b: (b, 0)),
        ],
        out_shape=[jax.ShapeDtypeStruct((_NPAD, hdim), jnp.float32)] * 2,
        scratch_shapes=[pltpu.VMEM((_TW * _BLK, g4), jnp.float32)],
    )(scal, panel, degcol, h, c, wih, whh, brow, xbrow)


def _tc_fc(h1, cbrow, w1, b1row, w2, b2row):
    def body(h_ref, cb_ref, w1_ref, b1_ref, w2_ref, b2_ref, o_ref):
        hb = h_ref[...] + cb_ref[...]
        a = jnp.dot(hb, w1_ref[...], preferred_element_type=jnp.float32) + b1_ref[...]
        a = jnp.maximum(a, 0.0)
        o_ref[...] = (jnp.dot(a, w2_ref[...], preferred_element_type=jnp.float32)
                      + b2_ref[...])

    return pl.pallas_call(
        body,
        grid=(_NB,),
        in_specs=[
            pl.BlockSpec((_BLK, _H1), lambda b: (b, 0)),
            pl.BlockSpec((1, _H1), lambda b: (0, 0)),
            pl.BlockSpec((_H1, _FCH), lambda b: (0, 0)),
            pl.BlockSpec((1, _FCH), lambda b: (0, 0)),
            pl.BlockSpec((_FCH, 128), lambda b: (0, 0)),
            pl.BlockSpec((1, 128), lambda b: (0, 0)),
        ],
        out_specs=pl.BlockSpec((_BLK, 128), lambda b: (b, 0)),
        out_shape=jax.ShapeDtypeStruct((_NPAD, 128), jnp.float32),
    )(h1, cbrow, w1, b1row, w2, b2row)


def kernel(x, node_ts, edge_index, conv0_W_ih, conv0_W_hh, conv0_b_ih,
           conv0_b_hh, conv0_bias, conv1_W_ih, conv1_W_hh, conv1_b_ih,
           conv1_b_hh, conv1_bias, fc1_W, fc1_b, fc2_W, fc2_b):
    f32 = jnp.float32
    src, dst = edge_index[0], edge_index[1]

    pad_e = _EPAD - _E
    src_p = jnp.concatenate([src.astype(jnp.int32),
                             jnp.zeros((pad_e,), jnp.int32)])
    dst_p = jnp.concatenate([dst.astype(jnp.int32),
                             jnp.full((pad_e,), _NKEY - 1, jnp.int32)])
    ts_pad = jnp.concatenate([node_ts.astype(jnp.int32),
                              jnp.zeros((_NKEY - _N,), jnp.int32)])
    pos_p, deg_full, key_p = _sc_rank(ts_pad, src_p, dst_p)
    deg = deg_full[:_N]
    max_deg = jnp.maximum(jnp.max(deg), 1)
    blocki = key_p // _BLK
    dloc = key_p % _BLK
    junk_tgt = _PROWS + (jnp.arange(_EPAD, dtype=jnp.int32) % _JUNK)

    degcol = jnp.concatenate(
        [deg, jnp.zeros((_NPAD - _N,), jnp.int32)]).reshape(_NPAD, 1)

    def run_conv(table, d, hdim, wih, whh, brow, xbrow):
        h = jnp.zeros((_NPAD, hdim), f32)
        c = jnp.zeros((_NPAD, hdim), f32)

        def cond(st):
            w, _, _ = st
            return w * _TW < max_deg

        def body(st):
            w, h, c = st
            wbase = w * _TW
            in_win = (key_p < _N) & (pos_p >= wbase) & (pos_p < wbase + _TW)
            tgt = jnp.where(
                in_win,
                blocki * (_TW * _BLK) + (pos_p - wbase) * _BLK + dloc,
                junk_tgt)
            panel = _sc_build_panel(table, src_p, tgt, d)
            twc = jnp.minimum(max_deg - wbase, _TW)
            scal = jnp.stack([twc, wbase]).astype(jnp.int32)
            h, c = _tc_conv_window(panel, degcol, h, c, scal, wih, whh,
                                   brow, xbrow, d, hdim)
            return (w + 1, h, c)

        _, h, _ = lax.while_loop(cond, body, (jnp.int32(0), h, c))
        return h

    w0ih = conv0_W_ih.T
    w0hh = conv0_W_hh.T
    b0 = (conv0_b_ih + conv0_b_hh).reshape(1, 4 * _H0)
    xb0 = jnp.zeros((1, _DIN), f32)
    h0 = run_conv(x, _DIN, _H0, w0ih, w0hh, b0, xb0)

    w1ih = conv1_W_ih.T
    w1hh = conv1_W_hh.T
    b1 = (conv1_b_ih + conv1_b_hh).reshape(1, 4 * _H1)
    xb1 = conv0_bias.reshape(1, _H0)
    h1 = run_conv(h0, _H0, _H1, w1ih, w1hh, b1, xb1)

    w2pad = jnp.zeros((_FCH, 128), f32).at[:, :2].set(fc2_W.T)
    b2row = jnp.zeros((1, 128), f32).at[:, :2].set(fc2_b.reshape(1, 2))
    out = _tc_fc(h1, conv1_bias.reshape(1, _H1), fc1_W.T,
                 fc1_b.reshape(1, _FCH), w2pad, b2row)
    return out[:_N, :2]
